# speedup vs baseline: 1.3048x; 1.3048x over previous
import jax
import jax.numpy as jnp
from jax.experimental import pallas as pl
from jax.experimental.pallas import tpu as pltpu

H = 512
W = 512


def _mask_kernel(bounds_ref, o_ref):
    b = pl.program_id(0)
    xs = bounds_ref[b, 0]
    xe = bounds_ref[b, 1]
    ys = bounds_ref[b, 2]
    ye = bounds_ref[b, 3]
    iy = jax.lax.broadcasted_iota(jnp.int32, (H, 1), 0)
    ix = jax.lax.broadcasted_iota(jnp.int32, (1, W), 1)
    row_f = ((iy >= ys) & (iy <= ye)).astype(jnp.float32)
    col_f = ((ix >= xs) & (ix <= xe)).astype(jnp.float32)
    o_ref[0] = row_f * col_f


def kernel(motion):
    B = motion.shape[0]
    xs = jnp.ceil(jnp.float32(H // 4) + motion[:, 0]).astype(jnp.int32)
    xe = jnp.ceil(jnp.float32(3 * H // 4) + motion[:, 0]).astype(jnp.int32)
    ys = jnp.ceil(jnp.float32(W // 4) + motion[:, 1]).astype(jnp.int32)
    ye = jnp.ceil(jnp.float32(3 * W // 4) + motion[:, 1]).astype(jnp.int32)
    bounds = jnp.stack([xs, xe, ys, ye], axis=1)

    out = pl.pallas_call(
        _mask_kernel,
        grid=(B,),
        in_specs=[pl.BlockSpec(memory_space=pltpu.SMEM)],
        out_specs=pl.BlockSpec((1, H, W), lambda b: (b, 0, 0)),
        out_shape=jax.ShapeDtypeStruct((B, H, W), jnp.float32),
    )(bounds)
    return out[..., None]


# speedup vs baseline: 1.5067x; 1.1548x over previous
import jax
import jax.numpy as jnp
from jax.experimental import pallas as pl
from jax.experimental.pallas import tpu as pltpu

H = 512
W = 512


G = 8


def _mask_kernel(bounds_ref, o_ref):
    g = pl.program_id(0)
    iy = jax.lax.broadcasted_iota(jnp.int32, (H, 1), 0)
    ix = jax.lax.broadcasted_iota(jnp.int32, (1, W), 1)
    for i in range(G):
        b = g * G + i
        xs = bounds_ref[b, 0]
        xe = bounds_ref[b, 1]
        ys = bounds_ref[b, 2]
        ye = bounds_ref[b, 3]
        row_f = ((iy >= ys) & (iy <= ye)).astype(jnp.float32)
        col_f = ((ix >= xs) & (ix <= xe)).astype(jnp.float32)
        o_ref[i] = row_f * col_f


def kernel(motion):
    B = motion.shape[0]
    xs = jnp.ceil(jnp.float32(H // 4) + motion[:, 0]).astype(jnp.int32)
    xe = jnp.ceil(jnp.float32(3 * H // 4) + motion[:, 0]).astype(jnp.int32)
    ys = jnp.ceil(jnp.float32(W // 4) + motion[:, 1]).astype(jnp.int32)
    ye = jnp.ceil(jnp.float32(3 * W // 4) + motion[:, 1]).astype(jnp.int32)
    bounds = jnp.stack([xs, xe, ys, ye], axis=1)

    out = pl.pallas_call(
        _mask_kernel,
        grid=(B // G,),
        in_specs=[pl.BlockSpec(memory_space=pltpu.SMEM)],
        out_specs=pl.BlockSpec((G, H, W), lambda g: (g, 0, 0)),
        out_shape=jax.ShapeDtypeStruct((B, H, W), jnp.float32),
    )(bounds)
    return out[..., None]
